# baseline (speedup 1.0000x reference)
import jax
import jax.numpy as jnp
from jax import lax
from jax.experimental import pallas as pl
from jax.experimental.pallas import tpu as pltpu
from jax.experimental.pallas import tpu_sc as plsc

BATCH = 4096
SEQ = 200
EMB_DIM = 128
NUM_WORKERS = 32
HALF_SEQ = SEQ // 2
NCHUNK = EMB_DIM // 16


def _bag_body(R, text_hbm, emb_hbm, out_hbm, idx_v, rows_v, out_stage, sem0, sem1, sem2):
    wid = lax.axis_index("s") * 2 + lax.axis_index("c")
    base = wid * R
    sems = (sem0, sem1, sem2)

    pltpu.sync_copy(text_hbm.at[pl.ds(base, R)], idx_v)

    def issue_half(r, b, h):
        pltpu.async_copy(emb_hbm.at[idx_v.at[r, h]], rows_v.at[b, h], sems[b])

    def issue(r, b):
        issue_half(r, b, 0)
        issue_half(r, b, 1)

    def wait_half(r, b, h):
        pltpu.make_async_copy(emb_hbm.at[idx_v.at[r, h]], rows_v.at[b, h], sems[b]).wait()

    def accum_half(b, h, acc):
        def tok_step(t, acc):
            for u in range(4):
                acc = tuple(
                    acc[c] + rows_v[b, h, 4 * t + u, pl.ds(c * 16, 16)]
                    for c in range(NCHUNK)
                )
            return acc

        return lax.fori_loop(0, HALF_SEQ // 4, tok_step, acc)

    def store_row(r, acc):
        for c in range(NCHUNK):
            out_stage[r, pl.ds(c * 16, 16)] = acc[c]

    nl = (R - 1) // 3
    issue(0, 0)
    issue(1, 1)
    issue(2, 2)

    def body(g, _):
        for b in range(3):
            r = 3 * g + b
            glim = (R - 4 - b) // 3
            acc = tuple(jnp.zeros((16,), jnp.float32) for _ in range(NCHUNK))
            for h in range(2):
                wait_half(r, b, h)
                acc = accum_half(b, h, acc)
                if glim >= nl - 1:
                    issue_half(r + 3, b, h)
                else:
                    @pl.when(g <= glim)
                    def _():
                        issue_half(r + 3, b, h)
            store_row(r, acc)
        return 0

    lax.fori_loop(0, nl, body, 0)
    for r in range(3 * nl, R):
        acc = tuple(jnp.zeros((16,), jnp.float32) for _ in range(NCHUNK))
        for h in range(2):
            wait_half(r, r % 3, h)
            acc = accum_half(r % 3, h, acc)
        store_row(r, acc)
    pltpu.sync_copy(out_stage, out_hbm.at[pl.ds(base, R)])


def _embedding_bag(text3, emb):
    import functools
    nbatch = text3.shape[0]
    R = nbatch // NUM_WORKERS
    mesh = plsc.VectorSubcoreMesh(core_axis_name="c", subcore_axis_name="s")
    run = pl.kernel(
        functools.partial(_bag_body, R),
        out_type=jax.ShapeDtypeStruct((nbatch, EMB_DIM), jnp.float32),
        mesh=mesh,
        scratch_types=[
            pltpu.VMEM((R, 2, HALF_SEQ), jnp.int32),
            pltpu.VMEM((3, 2, HALF_SEQ, EMB_DIM), jnp.float32),
            pltpu.VMEM((R, EMB_DIM), jnp.float32),
            pltpu.SemaphoreType.DMA,
            pltpu.SemaphoreType.DMA,
            pltpu.SemaphoreType.DMA,
        ],
    )
    return run(text3, emb)


def _mlp_body(x_ref, w1_ref, b1_ref, w2_ref, b2_ref, wc_ref, bc_ref, out_ref):
    x = jnp.tanh(x_ref[...])
    h1 = jnp.tanh(jnp.dot(x, w1_ref[...].T, preferred_element_type=jnp.float32) + b1_ref[...])
    h2 = jnp.tanh(jnp.dot(h1, w2_ref[...].T, preferred_element_type=jnp.float32) + b2_ref[...])
    logits = jnp.dot(h2, wc_ref[...].T, preferred_element_type=jnp.float32) + bc_ref[...]
    m = jnp.max(logits, axis=-1, keepdims=True)
    e = jnp.exp(logits - m)
    out_ref[...] = e / jnp.sum(e, axis=-1, keepdims=True)


def _mlp(summed, W1, b1, W2, b2, Wc, bc):
    blk = 512
    grid = (summed.shape[0] // blk,)
    full = lambda shape: pl.BlockSpec(shape, lambda i: (0,) * len(shape))
    return pl.pallas_call(
        _mlp_body,
        grid=grid,
        in_specs=[
            pl.BlockSpec((blk, EMB_DIM), lambda i: (i, 0)),
            full(W1.shape),
            full(b1.shape),
            full(W2.shape),
            full(b2.shape),
            full(Wc.shape),
            full(bc.shape),
        ],
        out_specs=pl.BlockSpec((blk, 2), lambda i: (i, 0)),
        out_shape=jax.ShapeDtypeStruct((summed.shape[0], 2), jnp.float32),
    )(summed, W1, b1, W2, b2, Wc, bc)


def kernel(text, emb, W1, b1, W2, b2, Wc, bc):
    text3 = text.astype(jnp.int32).reshape(BATCH, 2, HALF_SEQ)
    b1r = b1.reshape(1, -1)
    b2r = b2.reshape(1, -1)
    bcr = bc.reshape(1, -1)
    summed = _embedding_bag(text3, emb)
    return _mlp(summed, W1, b1r, W2, b2r, Wc, bcr)


# speedup vs baseline: 1.0021x; 1.0021x over previous
import jax
import jax.numpy as jnp
from jax import lax
from jax.experimental import pallas as pl
from jax.experimental.pallas import tpu as pltpu
from jax.experimental.pallas import tpu_sc as plsc

BATCH = 4096
SEQ = 200
EMB_DIM = 128
NUM_WORKERS = 32
HALF_SEQ = SEQ // 2
NCHUNK = EMB_DIM // 16


def _bag_body(R, text_hbm, emb_hbm, out_hbm, idx_v, rows_v, out_stage, sem0, sem1, sem2):
    wid = lax.axis_index("s") * 2 + lax.axis_index("c")
    base = wid * R
    sems = (sem0, sem1, sem2)

    pltpu.sync_copy(text_hbm.at[pl.ds(base, R)], idx_v)

    def issue_half(r, b, h):
        pltpu.async_copy(emb_hbm.at[idx_v.at[r, h]], rows_v.at[b, h], sems[b])

    def issue(r, b):
        issue_half(r, b, 0)
        issue_half(r, b, 1)

    def wait_half(r, b, h):
        pltpu.make_async_copy(emb_hbm.at[idx_v.at[r, h]], rows_v.at[b, h], sems[b]).wait()

    def accum_half(b, h, acc):
        def tok_step(t, acc):
            for u in range(4):
                acc = tuple(
                    acc[c] + rows_v[b, h, 4 * t + u, pl.ds(c * 16, 16)]
                    for c in range(NCHUNK)
                )
            return acc

        return lax.fori_loop(0, HALF_SEQ // 4, tok_step, acc)

    def store_row(r, acc):
        for c in range(NCHUNK):
            out_stage[r, pl.ds(c * 16, 16)] = acc[c]

    nl = (R - 1) // 3
    issue(0, 0)
    issue(1, 1)
    issue(2, 2)

    def body(g, _):
        for b in range(3):
            r = 3 * g + b
            glim = (R - 4 - b) // 3
            wait_half(r, b, 0)
            wait_half(r, b, 1)
            acc = tuple(jnp.zeros((16,), jnp.float32) for _ in range(NCHUNK))
            acc = accum_half(b, 0, acc)
            acc = accum_half(b, 1, acc)
            store_row(r, acc)
            if glim >= nl - 1:
                issue(r + 3, b)
            else:
                @pl.when(g <= glim)
                def _():
                    issue(r + 3, b)
        return 0

    lax.fori_loop(0, nl, body, 0)
    for r in range(3 * nl, R):
        wait_half(r, r % 3, 0)
        wait_half(r, r % 3, 1)
        acc = tuple(jnp.zeros((16,), jnp.float32) for _ in range(NCHUNK))
        acc = accum_half(r % 3, 0, acc)
        acc = accum_half(r % 3, 1, acc)
        store_row(r, acc)
    pltpu.sync_copy(out_stage, out_hbm.at[pl.ds(base, R)])


def _embedding_bag(text3, emb):
    import functools
    nbatch = text3.shape[0]
    R = nbatch // NUM_WORKERS
    mesh = plsc.VectorSubcoreMesh(core_axis_name="c", subcore_axis_name="s")
    run = pl.kernel(
        functools.partial(_bag_body, R),
        out_type=jax.ShapeDtypeStruct((nbatch, EMB_DIM), jnp.float32),
        mesh=mesh,
        scratch_types=[
            pltpu.VMEM((R, 2, HALF_SEQ), jnp.int32),
            pltpu.VMEM((3, 2, HALF_SEQ, EMB_DIM), jnp.float32),
            pltpu.VMEM((R, EMB_DIM), jnp.float32),
            pltpu.SemaphoreType.DMA,
            pltpu.SemaphoreType.DMA,
            pltpu.SemaphoreType.DMA,
        ],
    )
    return run(text3, emb)


def _mlp_body(x_ref, w1_ref, b1_ref, w2_ref, b2_ref, wc_ref, bc_ref, out_ref):
    x = jnp.tanh(x_ref[...])
    h1 = jnp.tanh(jnp.dot(x, w1_ref[...].T, preferred_element_type=jnp.float32) + b1_ref[...])
    h2 = jnp.tanh(jnp.dot(h1, w2_ref[...].T, preferred_element_type=jnp.float32) + b2_ref[...])
    logits = jnp.dot(h2, wc_ref[...].T, preferred_element_type=jnp.float32) + bc_ref[...]
    m = jnp.max(logits, axis=-1, keepdims=True)
    e = jnp.exp(logits - m)
    out_ref[...] = e / jnp.sum(e, axis=-1, keepdims=True)


def _mlp(summed, W1, b1, W2, b2, Wc, bc):
    blk = 512
    grid = (summed.shape[0] // blk,)
    full = lambda shape: pl.BlockSpec(shape, lambda i: (0,) * len(shape))
    return pl.pallas_call(
        _mlp_body,
        grid=grid,
        in_specs=[
            pl.BlockSpec((blk, EMB_DIM), lambda i: (i, 0)),
            full(W1.shape),
            full(b1.shape),
            full(W2.shape),
            full(b2.shape),
            full(Wc.shape),
            full(bc.shape),
        ],
        out_specs=pl.BlockSpec((blk, 2), lambda i: (i, 0)),
        out_shape=jax.ShapeDtypeStruct((summed.shape[0], 2), jnp.float32),
    )(summed, W1, b1, W2, b2, Wc, bc)


def kernel(text, emb, W1, b1, W2, b2, Wc, bc):
    text3 = text.astype(jnp.int32).reshape(BATCH, 2, HALF_SEQ)
    b1r = b1.reshape(1, -1)
    b2r = b2.reshape(1, -1)
    bcr = bc.reshape(1, -1)
    summed = _embedding_bag(text3, emb)
    return _mlp(summed, W1, b1r, W2, b2r, Wc, bcr)
